# baseline (speedup 1.0000x reference)
import functools

import jax
import jax.numpy as jnp
from jax import lax
from jax.experimental import pallas as pl
from jax.experimental.pallas import tpu as pltpu
from jax.experimental.pallas import tpu_sc as plsc

VOCAB = 1000000
EMBED_DIM = 64
BATCH = 4096
SEQ = 200

NC = 2
NS = 16
NW = NC * NS

N_ROWS = BATCH * SEQ
PER_W = N_ROWS // NW
CH = 128
NCHUNK = PER_W // CH
NBUF = 4
NWB = 2

V_PAIR = VOCAB // 2
PK_BLK = 5000

SEQ_PAD = 256
BROWS = BATCH // NW
XST = 16


def _pack_tc_body(top_ref, bot_ref, out_ref):
    out_ref[:, :EMBED_DIM] = top_ref[...]
    out_ref[:, EMBED_DIM:] = bot_ref[...]


_pack_tc = pl.pallas_call(
    _pack_tc_body,
    grid=(V_PAIR // PK_BLK,),
    in_specs=[
        pl.BlockSpec((PK_BLK, EMBED_DIM), lambda i: (i, 0)),
        pl.BlockSpec((PK_BLK, EMBED_DIM), lambda i: (i + V_PAIR // PK_BLK, 0)),
    ],
    out_specs=pl.BlockSpec((PK_BLK, 2 * EMBED_DIM), lambda i: (i, 0)),
    out_shape=jax.ShapeDtypeStruct((V_PAIR, 2 * EMBED_DIM), jnp.float32),
)


@functools.partial(
    pl.kernel,
    out_type=jax.ShapeDtypeStruct((N_ROWS, 2 * EMBED_DIM), jnp.float32),
    mesh=plsc.VectorSubcoreMesh(core_axis_name="c", subcore_axis_name="s"),
    scratch_types=[
        pltpu.VMEM((PER_W + XST,), jnp.int32),
        pltpu.VMEM((XST, SEQ_PAD), jnp.int32),
        pltpu.VMEM((NBUF, CH), jnp.int32),
        pltpu.VMEM((NBUF, CH, 2 * EMBED_DIM), jnp.float32),
        pltpu.VMEM((NWB, CH, 2 * EMBED_DIM), jnp.float32),
    ]
    + [pltpu.SemaphoreType.DMA] * NBUF
    + [pltpu.SemaphoreType.DMA] * NWB,
    compiler_params=pltpu.CompilerParams(use_tc_tiling_on_sc=True),
)
def _gather_kernel(idx_hbm, table_hbm, out_hbm, idx_v, raw_v, hi_v, rows_v,
                   outb_v, *sems):
    gsem = sems[:NBUF]
    wsem = sems[NBUF:]
    wid = lax.axis_index("s") * NC + lax.axis_index("c")
    base = wid * PER_W

    def stage(t, carry):
        pltpu.sync_copy(idx_hbm.at[pl.ds(wid * BROWS + t * XST, XST)], raw_v)

        def row(bb, carry2):
            f0 = (t * XST + bb) * SEQ
            for m in range(SEQ // 16):
                idx_v[pl.ds(f0 + m * 16, 16)] = raw_v[bb, pl.ds(m * 16, 16)]
            idx_v[pl.ds(f0 + SEQ - 16, 16)] = (
                raw_v[bb, pl.ds(SEQ - 16, 16)])
            return carry2

        lax.fori_loop(0, XST, row, 0)
        return carry

    lax.fori_loop(0, BROWS // XST, stage, 0)

    def compute_hi(j, b):
        for m in range(CH // 16):
            v = idx_v[pl.ds(j * CH + m * 16, 16)]
            wrap = lax.shift_right_logical(V_PAIR - 1 - v, 31)
            hi_v[b, pl.ds(m * 16, 16)] = v - wrap * V_PAIR

    def fire_gather(b):
        pltpu.async_copy(table_hbm.at[hi_v.at[b]], rows_v.at[b], gsem[b])

    def wait_gather(b):
        pltpu.make_async_copy(
            table_hbm.at[hi_v.at[b]], rows_v.at[b], gsem[b]).wait()

    def fire_write(j, w):
        pltpu.async_copy(
            outb_v.at[w], out_hbm.at[pl.ds(base + j * CH, CH)], wsem[w])

    def wait_write(w):
        pltpu.make_async_copy(
            outb_v.at[w], out_hbm.at[pl.ds(base, CH)], wsem[w]).wait()

    def select(j, b, w):
        rows = rows_v.at[b]
        outb = outb_v.at[w]

        def grp(g, carry):
            pv = lax.shift_right_logical(
                V_PAIR - 1 - idx_v[pl.ds(j * CH + g * 16, 16)], 31)
            for l in range(16):
                i = g * 16 + l
                p = pv[l]
                src0 = p * EMBED_DIM
                for k in range(EMBED_DIM // 16):
                    outb[i, pl.ds(k * 16, 16)] = (
                        rows[i, pl.ds(src0 + k * 16, 16)])
            return carry

        lax.fori_loop(0, CH // 16, grp, 0)

    for b in range(NBUF):
        compute_hi(b, b)
        fire_gather(b)

    def group(t, carry):
        for b in range(NBUF):
            j = t * NBUF + b
            w = b % NWB
            wait_gather(b)

            if b >= NWB:
                wait_write(w)
            else:
                @pl.when(t > 0)
                def _():
                    wait_write(w)

            select(j, b, w)
            fire_write(j, w)

            @pl.when(t < NCHUNK // NBUF - 1)
            def _():
                compute_hi(j + NBUF, b)
                fire_gather(b)

        return carry

    lax.fori_loop(0, NCHUNK // NBUF, group, 0)

    for w in range(NWB):
        wait_write(w)


SL_BLK = 2048


def _slice_tc_body(in_ref, out_ref):
    out_ref[...] = in_ref[:, :EMBED_DIM]


_slice_tc = pl.pallas_call(
    _slice_tc_body,
    grid=(N_ROWS // SL_BLK,),
    in_specs=[pl.BlockSpec((SL_BLK, 2 * EMBED_DIM), lambda i: (i, 0))],
    out_specs=pl.BlockSpec((SL_BLK, EMBED_DIM), lambda i: (i, 0)),
    out_shape=jax.ShapeDtypeStruct((N_ROWS, EMBED_DIM), jnp.float32),
)


def kernel(x, embedding_weight):
    xp = jnp.pad(x, ((0, 0), (0, SEQ_PAD - SEQ)))
    packed_tbl = _pack_tc(embedding_weight, embedding_weight)
    wide = _gather_kernel(xp, packed_tbl)
    return _slice_tc(wide).reshape(BATCH, SEQ, EMBED_DIM)


# speedup vs baseline: 1.3440x; 1.3440x over previous
import functools

import jax
import jax.numpy as jnp
from jax import lax
from jax.experimental import pallas as pl
from jax.experimental.pallas import tpu as pltpu
from jax.experimental.pallas import tpu_sc as plsc

VOCAB = 1000000
EMBED_DIM = 64
BATCH = 4096
SEQ = 200

NC = 2
NS = 16
NW = NC * NS

N_ROWS = BATCH * SEQ
PER_W = N_ROWS // NW
CH = 128
NCHUNK = PER_W // CH
NBUF = 4
NWB = 2

V_PAIR = VOCAB // 2
PK_BLK = 5000

SEQ_PAD = 256
BROWS = BATCH // NW
XST = 16


def _pack_tc_body(top_ref, bot_ref, out_ref):
    out_ref[:, :EMBED_DIM] = top_ref[...]
    out_ref[:, EMBED_DIM:] = bot_ref[...]


_pack_tc = pl.pallas_call(
    _pack_tc_body,
    grid=(V_PAIR // PK_BLK,),
    in_specs=[
        pl.BlockSpec((PK_BLK, EMBED_DIM), lambda i: (i, 0)),
        pl.BlockSpec((PK_BLK, EMBED_DIM), lambda i: (i + V_PAIR // PK_BLK, 0)),
    ],
    out_specs=pl.BlockSpec((PK_BLK, 2 * EMBED_DIM), lambda i: (i, 0)),
    out_shape=jax.ShapeDtypeStruct((V_PAIR, 2 * EMBED_DIM), jnp.float32),
)


@functools.partial(
    pl.kernel,
    out_type=jax.ShapeDtypeStruct((N_ROWS, 2 * EMBED_DIM), jnp.float32),
    mesh=plsc.VectorSubcoreMesh(core_axis_name="c", subcore_axis_name="s"),
    scratch_types=[
        pltpu.VMEM((PER_W + XST,), jnp.int32),
        pltpu.VMEM((XST, SEQ_PAD), jnp.int32),
        pltpu.VMEM((NBUF, CH), jnp.int32),
        pltpu.VMEM((NBUF, CH, 2 * EMBED_DIM), jnp.float32),
        pltpu.VMEM((NWB, CH, 2 * EMBED_DIM), jnp.float32),
    ]
    + [pltpu.SemaphoreType.DMA] * NBUF
    + [pltpu.SemaphoreType.DMA] * NWB,
    compiler_params=pltpu.CompilerParams(use_tc_tiling_on_sc=True),
)
def _gather_kernel(idx_hbm, table_hbm, out_hbm, idx_v, raw_v, hi_v, rows_v,
                   outb_v, *sems):
    gsem = sems[:NBUF]
    wsem = sems[NBUF:]
    wid = lax.axis_index("s") * NC + lax.axis_index("c")
    base = wid * PER_W

    def stage(t, carry):
        pltpu.sync_copy(idx_hbm.at[pl.ds(wid * BROWS + t * XST, XST)], raw_v)

        def row(bb, carry2):
            f0 = (t * XST + bb) * SEQ
            for m in range(SEQ // 16):
                idx_v[pl.ds(f0 + m * 16, 16)] = raw_v[bb, pl.ds(m * 16, 16)]
            idx_v[pl.ds(f0 + SEQ - 16, 16)] = (
                raw_v[bb, pl.ds(SEQ - 16, 16)])
            return carry2

        lax.fori_loop(0, XST, row, 0)
        return carry

    lax.fori_loop(0, BROWS // XST, stage, 0)

    def compute_hi(j, b):
        for m in range(CH // 16):
            v = idx_v[pl.ds(j * CH + m * 16, 16)]
            wrap = lax.shift_right_logical(V_PAIR - 1 - v, 31)
            hi_v[b, pl.ds(m * 16, 16)] = v - wrap * V_PAIR

    def fire_gather(b):
        pltpu.async_copy(table_hbm.at[hi_v.at[b]], rows_v.at[b], gsem[b])

    def wait_gather(b):
        pltpu.make_async_copy(
            table_hbm.at[hi_v.at[b]], rows_v.at[b], gsem[b]).wait()

    def fire_write(j, w):
        pltpu.async_copy(
            outb_v.at[w], out_hbm.at[pl.ds(base + j * CH, CH)], wsem[w])

    def wait_write(w):
        pltpu.make_async_copy(
            outb_v.at[w], out_hbm.at[pl.ds(base, CH)], wsem[w]).wait()

    def select(j, b, w):
        rows = rows_v.at[b]
        outb = outb_v.at[w]

        def grp(g, carry):
            pv = lax.shift_right_logical(
                V_PAIR - 1 - idx_v[pl.ds(j * CH + g * 16, 16)], 31)
            for l in range(16):
                i = g * 16 + l
                p = pv[l]
                src0 = p * EMBED_DIM
                for k in range(EMBED_DIM // 16):
                    outb[i, pl.ds(k * 16, 16)] = (
                        rows[i, pl.ds(src0 + k * 16, 16)])
            return carry

        lax.fori_loop(0, CH // 16, grp, 0)

    for b in range(NBUF):
        compute_hi(b, b)
        fire_gather(b)

    def group(t, carry):
        for b in range(NBUF):
            j = t * NBUF + b
            w = b % NWB
            wait_gather(b)

            if b >= NWB:
                wait_write(w)
            else:
                @pl.when(t > 0)
                def _():
                    wait_write(w)

            select(j, b, w)
            fire_write(j, w)

            @pl.when(t < NCHUNK // NBUF - 1)
            def _():
                compute_hi(j + NBUF, b)
                fire_gather(b)

        return carry

    lax.fori_loop(0, NCHUNK // NBUF, group, 0)

    for w in range(NWB):
        wait_write(w)


def kernel(x, embedding_weight):
    xp = jnp.pad(x, ((0, 0), (0, SEQ_PAD - SEQ)))
    packed_tbl = _pack_tc(embedding_weight, embedding_weight)
    wide = _gather_kernel(xp, packed_tbl)
    return wide[:, :EMBED_DIM].reshape(BATCH, SEQ, EMBED_DIM)


# speedup vs baseline: 1.4554x; 1.0829x over previous
import functools

import jax
import jax.numpy as jnp
from jax import lax
from jax.experimental import pallas as pl
from jax.experimental.pallas import tpu as pltpu
from jax.experimental.pallas import tpu_sc as plsc

VOCAB = 1000000
EMBED_DIM = 64
BATCH = 4096
SEQ = 200

NC = 2
NS = 16
NW = NC * NS

N_ROWS = BATCH * SEQ
PER_W = N_ROWS // NW
CH = 128
NCHUNK = PER_W // CH
NBUF = 4
NWB = 2

V_PAIR = VOCAB // 2
PK_BLK = 5000

SEQ_PAD = 256
BROWS = BATCH // NW
XST = 16


def _pack_tc_body(top_ref, bot_ref, out_ref):
    out_ref[:, :EMBED_DIM] = top_ref[...]
    out_ref[:, EMBED_DIM:] = bot_ref[...]


_pack_tc = pl.pallas_call(
    _pack_tc_body,
    grid=(V_PAIR // PK_BLK,),
    in_specs=[
        pl.BlockSpec((PK_BLK, EMBED_DIM), lambda i: (i, 0)),
        pl.BlockSpec((PK_BLK, EMBED_DIM), lambda i: (i + V_PAIR // PK_BLK, 0)),
    ],
    out_specs=pl.BlockSpec((PK_BLK, 2 * EMBED_DIM), lambda i: (i, 0)),
    out_shape=jax.ShapeDtypeStruct((V_PAIR, 2 * EMBED_DIM), jnp.float32),
)


@functools.partial(
    pl.kernel,
    out_type=jax.ShapeDtypeStruct((N_ROWS, 2 * EMBED_DIM), jnp.float32),
    mesh=plsc.VectorSubcoreMesh(core_axis_name="c", subcore_axis_name="s"),
    scratch_types=[
        pltpu.VMEM((PER_W + XST,), jnp.int32),
        pltpu.VMEM((XST, SEQ_PAD), jnp.int32),
        pltpu.VMEM((NBUF, CH), jnp.int32),
        pltpu.VMEM((NBUF, CH, 2 * EMBED_DIM), jnp.float32),
        pltpu.VMEM((NWB, CH, 2 * EMBED_DIM), jnp.float32),
    ]
    + [pltpu.SemaphoreType.DMA] * NBUF
    + [pltpu.SemaphoreType.DMA] * NWB,
    compiler_params=pltpu.CompilerParams(use_tc_tiling_on_sc=True),
)
def _gather_kernel(idx_hbm, table_hbm, out_hbm, idx_v, raw_v, hi_v, rows_v,
                   outb_v, *sems):
    gsem = sems[:NBUF]
    wsem = sems[NBUF:]
    wid = lax.axis_index("s") * NC + lax.axis_index("c")
    base = wid * PER_W

    def stage(t, carry):
        pltpu.sync_copy(idx_hbm.at[pl.ds(wid * BROWS + t * XST, XST)], raw_v)

        def row(bb, carry2):
            f0 = (t * XST + bb) * SEQ
            for m in range(SEQ // 16):
                idx_v[pl.ds(f0 + m * 16, 16)] = raw_v[bb, pl.ds(m * 16, 16)]
            idx_v[pl.ds(f0 + SEQ - 16, 16)] = (
                raw_v[bb, pl.ds(SEQ - 16, 16)])
            return carry2

        lax.fori_loop(0, XST, row, 0)
        return carry

    lax.fori_loop(0, BROWS // XST, stage, 0)

    def compute_hi(j, b):
        for m in range(CH // 16):
            v = idx_v[pl.ds(j * CH + m * 16, 16)]
            wrap = lax.shift_right_logical(V_PAIR - 1 - v, 31)
            hi_v[b, pl.ds(m * 16, 16)] = v - wrap * V_PAIR

    def fire_gather(b):
        pltpu.async_copy(table_hbm.at[hi_v.at[b]], rows_v.at[b], gsem[b])

    def wait_gather(b):
        pltpu.make_async_copy(
            table_hbm.at[hi_v.at[b]], rows_v.at[b], gsem[b]).wait()

    def fire_write(j, w):
        pltpu.async_copy(
            outb_v.at[w], out_hbm.at[pl.ds(base + j * CH, CH)], wsem[w])

    def wait_write(w):
        pltpu.make_async_copy(
            outb_v.at[w], out_hbm.at[pl.ds(base, CH)], wsem[w]).wait()

    def select(j, b, w):
        rows = rows_v.at[b]
        outb = outb_v.at[w]

        @plsc.parallel_loop(0, CH // 16, unroll=2)
        def _grp(g):
            pv = lax.shift_right_logical(
                V_PAIR - 1 - idx_v[pl.ds(j * CH + g * 16, 16)], 31)
            for l in range(16):
                i = g * 16 + l
                p = pv[l]
                src0 = p * EMBED_DIM
                for k in range(EMBED_DIM // 16):
                    outb[i, pl.ds(k * 16, 16)] = (
                        rows[i, pl.ds(src0 + k * 16, 16)])

    for b in range(NBUF):
        compute_hi(b, b)
        fire_gather(b)

    def group(t, carry):
        for b in range(NBUF):
            j = t * NBUF + b
            w = b % NWB
            wait_gather(b)

            if b >= NWB:
                wait_write(w)
            else:
                @pl.when(t > 0)
                def _():
                    wait_write(w)

            select(j, b, w)
            fire_write(j, w)

            @pl.when(t < NCHUNK // NBUF - 1)
            def _():
                compute_hi(j + NBUF, b)
                fire_gather(b)

        return carry

    lax.fori_loop(0, NCHUNK // NBUF, group, 0)

    for w in range(NWB):
        wait_write(w)


def kernel(x, embedding_weight):
    xp = jnp.pad(x, ((0, 0), (0, SEQ_PAD - SEQ)))
    packed_tbl = _pack_tc(embedding_weight, embedding_weight)
    wide = _gather_kernel(xp, packed_tbl)
    return wide[:, :EMBED_DIM].reshape(BATCH, SEQ, EMBED_DIM)


# speedup vs baseline: 1.4556x; 1.0002x over previous
import functools

import jax
import jax.numpy as jnp
from jax import lax
from jax.experimental import pallas as pl
from jax.experimental.pallas import tpu as pltpu
from jax.experimental.pallas import tpu_sc as plsc

VOCAB = 1000000
EMBED_DIM = 64
BATCH = 4096
SEQ = 200

NC = 2
NS = 16
NW = NC * NS

N_ROWS = BATCH * SEQ
PER_W = N_ROWS // NW
CH = 128
NCHUNK = PER_W // CH
NBUF = 4
NWB = 2

V_PAIR = VOCAB // 2
PK_BLK = 5000

SEQ_PAD = 256
BROWS = BATCH // NW
XST = 16


def _pack_tc_body(top_ref, bot_ref, out_ref):
    out_ref[:, :EMBED_DIM] = top_ref[...]
    out_ref[:, EMBED_DIM:] = bot_ref[...]


_pack_tc = pl.pallas_call(
    _pack_tc_body,
    grid=(V_PAIR // PK_BLK,),
    in_specs=[
        pl.BlockSpec((PK_BLK, EMBED_DIM), lambda i: (i, 0)),
        pl.BlockSpec((PK_BLK, EMBED_DIM), lambda i: (i + V_PAIR // PK_BLK, 0)),
    ],
    out_specs=pl.BlockSpec((PK_BLK, 2 * EMBED_DIM), lambda i: (i, 0)),
    out_shape=jax.ShapeDtypeStruct((V_PAIR, 2 * EMBED_DIM), jnp.float32),
)


@functools.partial(
    pl.kernel,
    out_type=jax.ShapeDtypeStruct((N_ROWS, 2 * EMBED_DIM), jnp.float32),
    mesh=plsc.VectorSubcoreMesh(core_axis_name="c", subcore_axis_name="s"),
    scratch_types=[
        pltpu.VMEM((PER_W + XST,), jnp.int32),
        pltpu.VMEM((XST, SEQ_PAD), jnp.int32),
        pltpu.VMEM((NBUF, CH), jnp.int32),
        pltpu.VMEM((NBUF, CH, 2 * EMBED_DIM), jnp.float32),
        pltpu.VMEM((NWB, CH, 2 * EMBED_DIM), jnp.float32),
    ]
    + [pltpu.SemaphoreType.DMA] * NBUF
    + [pltpu.SemaphoreType.DMA] * NWB,
    compiler_params=pltpu.CompilerParams(use_tc_tiling_on_sc=True),
)
def _gather_kernel(idx_hbm, table_hbm, out_hbm, idx_v, raw_v, hi_v, rows_v,
                   outb_v, *sems):
    gsem = sems[:NBUF]
    wsem = sems[NBUF:]
    wid = lax.axis_index("s") * NC + lax.axis_index("c")
    base = wid * PER_W

    def stage(t, carry):
        pltpu.sync_copy(idx_hbm.at[pl.ds(wid * BROWS + t * XST, XST)], raw_v)

        @plsc.parallel_loop(0, XST, unroll=2)
        def _row(bb):
            f0 = (t * XST + bb) * SEQ
            for m in range(SEQ // 16):
                idx_v[pl.ds(f0 + m * 16, 16)] = raw_v[bb, pl.ds(m * 16, 16)]
            idx_v[pl.ds(f0 + SEQ - 16, 16)] = (
                raw_v[bb, pl.ds(SEQ - 16, 16)])

        return carry

    lax.fori_loop(0, BROWS // XST, stage, 0)

    def compute_hi(j, b):
        for m in range(CH // 16):
            v = idx_v[pl.ds(j * CH + m * 16, 16)]
            wrap = lax.shift_right_logical(V_PAIR - 1 - v, 31)
            hi_v[b, pl.ds(m * 16, 16)] = v - wrap * V_PAIR

    def fire_gather(b):
        pltpu.async_copy(table_hbm.at[hi_v.at[b]], rows_v.at[b], gsem[b])

    def wait_gather(b):
        pltpu.make_async_copy(
            table_hbm.at[hi_v.at[b]], rows_v.at[b], gsem[b]).wait()

    def fire_write(j, w):
        pltpu.async_copy(
            outb_v.at[w], out_hbm.at[pl.ds(base + j * CH, CH)], wsem[w])

    def wait_write(w):
        pltpu.make_async_copy(
            outb_v.at[w], out_hbm.at[pl.ds(base, CH)], wsem[w]).wait()

    def select(j, b, w):
        rows = rows_v.at[b]
        outb = outb_v.at[w]

        @plsc.parallel_loop(0, CH // 16, unroll=4)
        def _grp(g):
            pv = lax.shift_right_logical(
                V_PAIR - 1 - idx_v[pl.ds(j * CH + g * 16, 16)], 31)
            for l in range(16):
                i = g * 16 + l
                p = pv[l]
                src0 = p * EMBED_DIM
                for k in range(EMBED_DIM // 16):
                    outb[i, pl.ds(k * 16, 16)] = (
                        rows[i, pl.ds(src0 + k * 16, 16)])

    for b in range(NBUF):
        compute_hi(b, b)
        fire_gather(b)

    def group(t, carry):
        for b in range(NBUF):
            j = t * NBUF + b
            w = b % NWB
            wait_gather(b)

            if b >= NWB:
                wait_write(w)
            else:
                @pl.when(t > 0)
                def _():
                    wait_write(w)

            select(j, b, w)
            fire_write(j, w)

            @pl.when(t < NCHUNK // NBUF - 1)
            def _():
                compute_hi(j + NBUF, b)
                fire_gather(b)

        return carry

    lax.fori_loop(0, NCHUNK // NBUF, group, 0)

    for w in range(NWB):
        wait_write(w)


def kernel(x, embedding_weight):
    xp = jnp.pad(x, ((0, 0), (0, SEQ_PAD - SEQ)))
    packed_tbl = _pack_tc(embedding_weight, embedding_weight)
    wide = _gather_kernel(xp, packed_tbl)
    return wide[:, :EMBED_DIM].reshape(BATCH, SEQ, EMBED_DIM)


# speedup vs baseline: 1.4565x; 1.0006x over previous
import functools

import jax
import jax.numpy as jnp
from jax import lax
from jax.experimental import pallas as pl
from jax.experimental.pallas import tpu as pltpu
from jax.experimental.pallas import tpu_sc as plsc

VOCAB = 1000000
EMBED_DIM = 64
BATCH = 4096
SEQ = 200

NC = 2
NS = 16
NW = NC * NS

N_ROWS = BATCH * SEQ
PER_W = N_ROWS // NW
CH = 128
NCHUNK = PER_W // CH
NBUF = 4
NWB = 2

V_PAIR = VOCAB // 2
PK_BLK = 5000

SEQ_PAD = 256
BROWS = BATCH // NW
XST = 16


def _pack_tc_body(top_ref, bot_ref, out_ref):
    out_ref[:, :EMBED_DIM] = top_ref[...]
    out_ref[:, EMBED_DIM:] = bot_ref[...]


_pack_tc = pl.pallas_call(
    _pack_tc_body,
    grid=(V_PAIR // PK_BLK,),
    in_specs=[
        pl.BlockSpec((PK_BLK, EMBED_DIM), lambda i: (i, 0)),
        pl.BlockSpec((PK_BLK, EMBED_DIM), lambda i: (i + V_PAIR // PK_BLK, 0)),
    ],
    out_specs=pl.BlockSpec((PK_BLK, 2 * EMBED_DIM), lambda i: (i, 0)),
    out_shape=jax.ShapeDtypeStruct((V_PAIR, 2 * EMBED_DIM), jnp.float32),
)


@functools.partial(
    pl.kernel,
    out_type=jax.ShapeDtypeStruct((N_ROWS, 2 * EMBED_DIM), jnp.float32),
    mesh=plsc.VectorSubcoreMesh(core_axis_name="c", subcore_axis_name="s"),
    scratch_types=[
        pltpu.VMEM((PER_W + XST,), jnp.int32),
        pltpu.VMEM((XST, SEQ_PAD), jnp.int32),
        pltpu.VMEM((NBUF, CH), jnp.int32),
        pltpu.VMEM((NBUF, CH, 2 * EMBED_DIM), jnp.float32),
        pltpu.VMEM((NWB, CH, 2 * EMBED_DIM), jnp.float32),
    ]
    + [pltpu.SemaphoreType.DMA] * NBUF
    + [pltpu.SemaphoreType.DMA] * NWB,
    compiler_params=pltpu.CompilerParams(use_tc_tiling_on_sc=True),
)
def _gather_kernel(idx_hbm, table_hbm, out_hbm, idx_v, raw_v, hi_v, rows_v,
                   outb_v, *sems):
    gsem = sems[:NBUF]
    wsem = sems[NBUF:]
    wid = lax.axis_index("s") * NC + lax.axis_index("c")
    base = wid * PER_W

    def stage(t, carry):
        pltpu.sync_copy(idx_hbm.at[pl.ds(wid * BROWS + t * XST, XST)], raw_v)

        @plsc.parallel_loop(0, XST, unroll=2)
        def _row(bb):
            f0 = (t * XST + bb) * SEQ
            for m in range(SEQ // 16):
                idx_v[pl.ds(f0 + m * 16, 16)] = raw_v[bb, pl.ds(m * 16, 16)]
            idx_v[pl.ds(f0 + SEQ - 16, 16)] = (
                raw_v[bb, pl.ds(SEQ - 16, 16)])

        return carry

    lax.fori_loop(0, BROWS // XST, stage, 0)

    def compute_hi(j, b):
        for m in range(CH // 16):
            v = idx_v[pl.ds(j * CH + m * 16, 16)]
            wrap = lax.shift_right_logical(V_PAIR - 1 - v, 31)
            hi_v[b, pl.ds(m * 16, 16)] = v - wrap * V_PAIR

    def fire_gather(b):
        pltpu.async_copy(table_hbm.at[hi_v.at[b]], rows_v.at[b], gsem[b])

    def wait_gather(b):
        pltpu.make_async_copy(
            table_hbm.at[hi_v.at[b]], rows_v.at[b], gsem[b]).wait()

    def fire_write(j, w):
        pltpu.async_copy(
            outb_v.at[w], out_hbm.at[pl.ds(base + j * CH, CH)], wsem[w])

    def wait_write(w):
        pltpu.make_async_copy(
            outb_v.at[w], out_hbm.at[pl.ds(base, CH)], wsem[w]).wait()

    def select(j, b, w):
        rows = rows_v.at[b]
        outb = outb_v.at[w]

        @plsc.parallel_loop(0, CH // 16, unroll=4)
        def _grp(g):
            pv = lax.shift_right_logical(
                V_PAIR - 1 - idx_v[pl.ds(j * CH + g * 16, 16)], 31)
            for l in range(16):
                i = g * 16 + l
                p = pv[l]
                src0 = p * EMBED_DIM
                for k in range(EMBED_DIM // 16):
                    outb[i, pl.ds(k * 16, 16)] = (
                        rows[i, pl.ds(src0 + k * 16, 16)])

    for b in range(NBUF):
        compute_hi(b, b)
        fire_gather(b)

    def group(t, carry):
        for b in range(NBUF):
            j = t * NBUF + b
            w = b % NWB
            wait_gather(b)

            if b >= NWB:
                wait_write(w)
            else:
                @pl.when(t > 0)
                def _():
                    wait_write(w)

            select(j, b, w)
            fire_write(j, w)

            @pl.when(t < NCHUNK // NBUF - 1)
            def _():
                compute_hi(j + NBUF, b)
                fire_gather(b)

        return carry

    lax.fori_loop(0, NCHUNK // NBUF, group, 0)

    for w in range(NWB):
        wait_write(w)


def kernel(x, embedding_weight):
    xp = jnp.pad(x, ((0, 0), (0, SEQ_PAD - SEQ)))
    packed_tbl = _pack_tc(embedding_weight, embedding_weight)
    wide = _gather_kernel(xp, packed_tbl)
    sliced = jnp.minimum(wide[:, :EMBED_DIM], jnp.inf)
    return sliced.reshape(BATCH, SEQ, EMBED_DIM)


# speedup vs baseline: 1.4590x; 1.0017x over previous
import functools

import jax
import jax.numpy as jnp
from jax import lax
from jax.experimental import pallas as pl
from jax.experimental.pallas import tpu as pltpu
from jax.experimental.pallas import tpu_sc as plsc

VOCAB = 1000000
EMBED_DIM = 64
BATCH = 4096
SEQ = 200

NC = 2
NS = 16
NW = NC * NS

N_ROWS = BATCH * SEQ
PER_W = N_ROWS // NW
CH = 128
NCHUNK = PER_W // CH
NBUF = 4
NWB = 2

V_PAIR = VOCAB // 2
PK_BLK = 5000

SEQ_PAD = 256
BROWS = BATCH // NW
XST = 16


def _pack_tc_body(top_ref, bot_ref, out_ref):
    out_ref[:, :EMBED_DIM] = top_ref[...]
    out_ref[:, EMBED_DIM:] = bot_ref[...]


_pack_tc = pl.pallas_call(
    _pack_tc_body,
    grid=(V_PAIR // PK_BLK,),
    in_specs=[
        pl.BlockSpec((PK_BLK, EMBED_DIM), lambda i: (i, 0)),
        pl.BlockSpec((PK_BLK, EMBED_DIM), lambda i: (i + V_PAIR // PK_BLK, 0)),
    ],
    out_specs=pl.BlockSpec((PK_BLK, 2 * EMBED_DIM), lambda i: (i, 0)),
    out_shape=jax.ShapeDtypeStruct((V_PAIR, 2 * EMBED_DIM), jnp.float32),
)


@functools.partial(
    pl.kernel,
    out_type=jax.ShapeDtypeStruct((N_ROWS, 2 * EMBED_DIM), jnp.float32),
    mesh=plsc.VectorSubcoreMesh(core_axis_name="c", subcore_axis_name="s"),
    scratch_types=[
        pltpu.VMEM((PER_W + XST,), jnp.int32),
        pltpu.VMEM((XST, SEQ_PAD), jnp.int32),
        pltpu.VMEM((NBUF, CH), jnp.int32),
        pltpu.VMEM((NBUF, CH, 2 * EMBED_DIM), jnp.float32),
        pltpu.VMEM((NWB, CH, 2 * EMBED_DIM), jnp.float32),
    ]
    + [pltpu.SemaphoreType.DMA] * NBUF
    + [pltpu.SemaphoreType.DMA] * NWB,
    compiler_params=pltpu.CompilerParams(use_tc_tiling_on_sc=True),
)
def _gather_kernel(idx_hbm, table_hbm, out_hbm, idx_v, raw_v, hi_v, rows_v,
                   outb_v, *sems):
    gsem = sems[:NBUF]
    wsem = sems[NBUF:]
    wid = lax.axis_index("s") * NC + lax.axis_index("c")
    base = wid * PER_W

    def stage(t, carry):
        pltpu.sync_copy(idx_hbm.at[pl.ds(wid * BROWS + t * XST, XST)], raw_v)

        @plsc.parallel_loop(0, XST, unroll=2)
        def _row(bb):
            f0 = (t * XST + bb) * SEQ
            for m in range(SEQ // 16):
                idx_v[pl.ds(f0 + m * 16, 16)] = raw_v[bb, pl.ds(m * 16, 16)]
            idx_v[pl.ds(f0 + SEQ - 16, 16)] = (
                raw_v[bb, pl.ds(SEQ - 16, 16)])

        return carry

    lax.fori_loop(0, BROWS // XST, stage, 0)

    def compute_hi(j, b):
        for m in range(CH // 16):
            v = idx_v[pl.ds(j * CH + m * 16, 16)]
            wrap = lax.shift_right_logical(V_PAIR - 1 - v, 31)
            hi_v[b, pl.ds(m * 16, 16)] = v - wrap * V_PAIR

    def fire_gather(b):
        pltpu.async_copy(table_hbm.at[hi_v.at[b]], rows_v.at[b], gsem[b])

    def wait_gather(b):
        pltpu.make_async_copy(
            table_hbm.at[hi_v.at[b]], rows_v.at[b], gsem[b]).wait()

    def fire_write(j, w):
        pltpu.async_copy(
            outb_v.at[w], out_hbm.at[pl.ds(base + j * CH, CH)], wsem[w])

    def wait_write(w):
        pltpu.make_async_copy(
            outb_v.at[w], out_hbm.at[pl.ds(base, CH)], wsem[w]).wait()

    def select(j, b, w):
        rows = rows_v.at[b]
        outb = outb_v.at[w]

        @plsc.parallel_loop(0, CH // 16, unroll=4)
        def _grp(g):
            pv = lax.shift_right_logical(
                V_PAIR - 1 - idx_v[pl.ds(j * CH + g * 16, 16)], 31)
            for l in range(16):
                i = g * 16 + l
                p = pv[l]
                src0 = p * EMBED_DIM
                for k in range(EMBED_DIM // 16):
                    outb[i, pl.ds(k * 16, 16)] = (
                        rows[i, pl.ds(src0 + k * 16, 16)])

    for b in range(NBUF):
        compute_hi(b, b)
        fire_gather(b)

    def group(t, carry):
        for b in range(NBUF):
            j = t * NBUF + b
            w = b % NWB
            wait_gather(b)

            if b >= NWB:
                wait_write(w)
            else:
                @pl.when(t > 0)
                def _():
                    wait_write(w)

            select(j, b, w)
            fire_write(j, w)

            @pl.when(t < NCHUNK // NBUF - 1)
            def _():
                compute_hi(j + NBUF, b)
                fire_gather(b)

        return carry

    lax.fori_loop(0, NCHUNK // NBUF, group, 0)

    for w in range(NWB):
        wait_write(w)


def kernel(x, embedding_weight):
    xp = jnp.pad(x, ((0, 0), (0, SEQ_PAD - SEQ)))
    packed_tbl = _pack_tc(embedding_weight, embedding_weight)
    wide = _gather_kernel(xp, packed_tbl)
    return wide[:, :EMBED_DIM].reshape(BATCH, SEQ, EMBED_DIM)
